# baseline (speedup 1.0000x reference)
import functools

import jax
import jax.numpy as jnp
from jax import lax
from jax.experimental import pallas as pl
from jax.experimental.pallas import tpu as pltpu
from jax.experimental.pallas import tpu_sc as plsc

_SRC_SHAPES = ((64, 64), (32, 32), (16, 16), (8, 8))
_STARTS = (0, 4096, 5120, 5376)
_PAD_SHAPES = tuple((h + 2, w + 2) for h, w in _SRC_SHAPES)
_PAD_STARTS = (0, 66 * 66, 66 * 66 + 34 * 34, 66 * 66 + 34 * 34 + 18 * 18)
_SP = 66 * 66 + 34 * 34 + 18 * 18 + 10 * 10

_B, _S, _H, _D = 2, 5440, 8, 32
_Q, _L, _P = 5440, 4, 4
_QB = 160
_NCH = _Q // _QB
_VB = _QB // 16
_NW = 32
_DH = 16


def _body(table_hbm, xy_hbm, aw_hbm, out_hbm, table_v, xy_v, aw_v, out_t, out_v):
    wid = lax.axis_index("s") * 2 + lax.axis_index("c")
    bh = wid // 2
    b = wid // (2 * _H)
    h = (wid // 2) % _H
    dh = wid % 2
    pltpu.sync_copy(table_hbm.at[wid], table_v)

    def chunk_body(ic, carry):
        q0 = ic * _QB
        pltpu.sync_copy(xy_hbm.at[bh, :, pl.ds(q0, _QB)], xy_v)
        pltpu.sync_copy(aw_hbm.at[bh, :, pl.ds(q0, _QB)], aw_v)

        def iv_body(iv, carry2):
            c0 = iv * 16
            acc = [jnp.zeros((16,), jnp.float32) for _ in range(_DH)]
            for l in range(_L):
                hh, ww = _SRC_SHAPES[l]
                wp = ww + 2
                base = _PAD_STARTS[l]
                for p in range(_P):
                    tx = xy_v[8 * l + 2 * p, pl.ds(c0, 16)] * float(ww) + 0.5
                    ty = xy_v[8 * l + 2 * p + 1, pl.ds(c0, 16)] * float(hh) + 0.5
                    w = aw_v[4 * l + p, pl.ds(c0, 16)]
                    x0 = tx.astype(jnp.int32)
                    y0 = ty.astype(jnp.int32)
                    fx = tx - x0.astype(jnp.float32)
                    fy = ty - y0.astype(jnp.float32)
                    ay1 = fy * w
                    ay0 = w - ay1
                    r0 = y0 * wp + base + x0
                    corners = (
                        (r0, (1.0 - fx) * ay0),
                        (r0 + 1, fx * ay0),
                        (r0 + wp, (1.0 - fx) * ay1),
                        (r0 + wp + 1, fx * ay1),
                    )
                    for adr, wc in corners:
                        cur = adr
                        for dd in range(_DH):
                            g = plsc.load_gather(table_v, [cur])
                            acc[dd] = acc[dd] + g * wc
                            if dd + 1 < _DH:
                                cur = cur + _SP
            for dd in range(_DH):
                out_t[dd, pl.ds(c0, 16)] = acc[dd]
            return carry2

        lax.fori_loop(0, _VB, iv_body, 0)

        def tr_body(iv, carry2):
            c0 = iv * 16
            qv = lax.iota(jnp.int32, 16) + c0
            for j in range(4):
                dv = jnp.full((16,), j * 4, jnp.int32)
                for k in range(4):
                    dd = j * 4 + k
                    v = out_t[dd, pl.ds(c0, 16)]
                    plsc.store_scatter(out_v, [qv, dv], v)
                    if k < 3:
                        dv = dv + 1
            return carry2

        lax.fori_loop(0, _VB, tr_body, 0)
        pltpu.sync_copy(out_v.at[:, pl.ds(0, _DH)],
                        out_hbm.at[b, pl.ds(q0, _QB),
                                   pl.ds((h * 2 + dh) * _DH, _DH)])
        return carry

    lax.fori_loop(0, _NCH, chunk_body, 0)


@jax.jit
def _run(table, xy, aw):
    mesh = plsc.VectorSubcoreMesh(core_axis_name="c", subcore_axis_name="s")
    f = functools.partial(
        pl.kernel,
        mesh=mesh,
        out_type=jax.ShapeDtypeStruct((_B, _Q, _H * _D), jnp.float32),
        scratch_types=[
            pltpu.VMEM((_DH * _SP,), jnp.float32),
            pltpu.VMEM((2 * _L * _P, _QB), jnp.float32),
            pltpu.VMEM((_L * _P, _QB), jnp.float32),
            pltpu.VMEM((_DH, _QB), jnp.float32),
            pltpu.VMEM((_QB, _DH + 1), jnp.float32),
        ],
        compiler_params=pltpu.CompilerParams(
            use_tc_tiling_on_sc=False, needs_layout_passes=False),
    )(_body)
    return f(table, xy, aw)


def kernel(value, value_spatial_shapes, value_level_start_index,
           sampling_locations, attention_weights, im2col_step):
    B, S, H, D = value.shape
    Q = sampling_locations.shape[1]
    vt = jnp.transpose(value, (0, 2, 3, 1)).reshape(B * H, D, S)
    parts = []
    for l, (hh, ww) in enumerate(_SRC_SHAPES):
        lvl = lax.slice_in_dim(vt, _STARTS[l], _STARTS[l] + hh * ww, axis=2)
        lvl = lvl.reshape(B * H, D, hh, ww)
        lvl = jnp.pad(lvl, ((0, 0), (0, 0), (1, 1), (1, 1)))
        parts.append(lvl.reshape(B * H, D, (hh + 2) * (ww + 2)))
    table = jnp.concatenate(parts, axis=2).reshape(_NW, _DH * _SP)
    xy = jnp.transpose(sampling_locations, (0, 2, 3, 4, 5, 1)).reshape(
        B * H, _L * _P * 2, Q)
    aw = jnp.transpose(attention_weights, (0, 2, 3, 4, 1)).reshape(
        B * H, _L * _P, Q)
    return _run(table, xy, aw)


# speedup vs baseline: 1.0899x; 1.0899x over previous
import functools

import jax
import jax.numpy as jnp
from jax import lax
from jax.experimental import pallas as pl
from jax.experimental.pallas import tpu as pltpu
from jax.experimental.pallas import tpu_sc as plsc

_SRC_SHAPES = ((64, 64), (32, 32), (16, 16), (8, 8))
_STARTS = (0, 4096, 5120, 5376)
_PAD_SHAPES = tuple((h + 2, w + 2) for h, w in _SRC_SHAPES)
_PAD_STARTS = (0, 66 * 66, 66 * 66 + 34 * 34, 66 * 66 + 34 * 34 + 18 * 18)
_SP = 66 * 66 + 34 * 34 + 18 * 18 + 10 * 10

_B, _S, _H, _D = 2, 5440, 8, 32
_Q, _L, _P = 5440, 4, 4
_QB = 272
_NCH = _Q // _QB
_VB = _QB // 16
_NW = 32
_DH = 16


def _body(table_hbm, xy_hbm, aw_hbm, out_hbm, table_v, xy_v, aw_v, out_v):
    wid = lax.axis_index("s") * 2 + lax.axis_index("c")
    bh = wid // 2
    pltpu.sync_copy(table_hbm.at[wid], table_v)

    def chunk_body(ic, carry):
        q0 = ic * _QB
        pltpu.sync_copy(xy_hbm.at[bh, :, pl.ds(q0, _QB)], xy_v)
        pltpu.sync_copy(aw_hbm.at[bh, :, pl.ds(q0, _QB)], aw_v)

        def iv_body(iv, carry2):
            c0 = iv * 16
            acc = [jnp.zeros((16,), jnp.float32) for _ in range(_DH)]
            for l in range(_L):
                hh, ww = _SRC_SHAPES[l]
                wp = ww + 2
                base = _PAD_STARTS[l]
                for p in range(_P):
                    tx = xy_v[8 * l + 2 * p, pl.ds(c0, 16)] * float(ww) + 0.5
                    ty = xy_v[8 * l + 2 * p + 1, pl.ds(c0, 16)] * float(hh) + 0.5
                    w = aw_v[4 * l + p, pl.ds(c0, 16)]
                    x0 = tx.astype(jnp.int32)
                    y0 = ty.astype(jnp.int32)
                    fx = tx - x0.astype(jnp.float32)
                    fy = ty - y0.astype(jnp.float32)
                    ay1 = fy * w
                    ay0 = w - ay1
                    r0 = y0 * wp + base + x0
                    corners = (
                        (r0, (1.0 - fx) * ay0),
                        (r0 + 1, fx * ay0),
                        (r0 + wp, (1.0 - fx) * ay1),
                        (r0 + wp + 1, fx * ay1),
                    )
                    for adr, wc in corners:
                        cur = adr
                        for dd in range(_DH):
                            g = plsc.load_gather(table_v, [cur])
                            acc[dd] = acc[dd] + g * wc
                            if dd + 1 < _DH:
                                cur = cur + _SP
            for dd in range(_DH):
                out_v[dd, pl.ds(c0, 16)] = acc[dd]
            return carry2

        lax.fori_loop(0, _VB, iv_body, 0)
        pltpu.sync_copy(out_v, out_hbm.at[wid, :, pl.ds(q0, _QB)])
        return carry

    lax.fori_loop(0, _NCH, chunk_body, 0)


@jax.jit
def _run(table, xy, aw):
    mesh = plsc.VectorSubcoreMesh(core_axis_name="c", subcore_axis_name="s")
    f = functools.partial(
        pl.kernel,
        mesh=mesh,
        out_type=jax.ShapeDtypeStruct((_NW, _DH, _Q), jnp.float32),
        scratch_types=[
            pltpu.VMEM((_DH * _SP,), jnp.float32),
            pltpu.VMEM((2 * _L * _P, _QB), jnp.float32),
            pltpu.VMEM((_L * _P, _QB), jnp.float32),
            pltpu.VMEM((_DH, _QB), jnp.float32),
        ],
        compiler_params=pltpu.CompilerParams(
            use_tc_tiling_on_sc=False, needs_layout_passes=False),
    )(_body)
    return f(table, xy, aw)


def kernel(value, value_spatial_shapes, value_level_start_index,
           sampling_locations, attention_weights, im2col_step):
    B, S, H, D = value.shape
    Q = sampling_locations.shape[1]
    vt = jnp.transpose(value, (0, 2, 3, 1)).reshape(B * H, D, S)
    parts = []
    for l, (hh, ww) in enumerate(_SRC_SHAPES):
        lvl = lax.slice_in_dim(vt, _STARTS[l], _STARTS[l] + hh * ww, axis=2)
        lvl = lvl.reshape(B * H, D, hh, ww)
        lvl = jnp.pad(lvl, ((0, 0), (0, 0), (1, 1), (1, 1)))
        parts.append(lvl.reshape(B * H, D, (hh + 2) * (ww + 2)))
    table = jnp.concatenate(parts, axis=2).reshape(_NW, _DH * _SP)
    xy = jnp.transpose(sampling_locations, (0, 2, 3, 4, 5, 1)).reshape(
        B * H, _L * _P * 2, Q)
    aw = jnp.transpose(attention_weights, (0, 2, 3, 4, 1)).reshape(
        B * H, _L * _P, Q)
    out = _run(table, xy, aw)
    out = out.reshape(B, H, D, Q)
    return jnp.transpose(out, (0, 3, 1, 2)).reshape(B, Q, H * D)


# speedup vs baseline: 1.1157x; 1.0237x over previous
import functools

import jax
import jax.numpy as jnp
from jax import lax
from jax.experimental import pallas as pl
from jax.experimental.pallas import tpu as pltpu
from jax.experimental.pallas import tpu_sc as plsc

_SRC_SHAPES = ((64, 64), (32, 32), (16, 16), (8, 8))
_STARTS = (0, 4096, 5120, 5376)
_PAD_SHAPES = tuple((h + 2, w + 2) for h, w in _SRC_SHAPES)
_PAD_STARTS = (0, 66 * 66, 66 * 66 + 34 * 34, 66 * 66 + 34 * 34 + 18 * 18)
_SP = 66 * 66 + 34 * 34 + 18 * 18 + 10 * 10

_B, _S, _H, _D = 2, 5440, 8, 32
_Q, _L, _P = 5440, 4, 4
_QB = 544
_NCH = _Q // _QB
_VB = _QB // 16
_NW = 32
_DH = 16


def _body(table_hbm, xy_hbm, aw_hbm, out_hbm, table_v, xy_v, aw_v, out_v):
    wid = lax.axis_index("s") * 2 + lax.axis_index("c")
    bh = wid // 2
    pltpu.sync_copy(table_hbm.at[wid], table_v)

    def chunk_body(ic, carry):
        q0 = ic * _QB
        pltpu.sync_copy(xy_hbm.at[bh, :, pl.ds(q0, _QB)], xy_v)
        pltpu.sync_copy(aw_hbm.at[bh, :, pl.ds(q0, _QB)], aw_v)

        def iv_body(iv, carry2):
            c0 = iv * 16
            acc = [jnp.zeros((16,), jnp.float32) for _ in range(_DH)]
            for l in range(_L):
                hh, ww = _SRC_SHAPES[l]
                wp = ww + 2
                base = _PAD_STARTS[l]
                for p in range(_P):
                    tx = xy_v[8 * l + 2 * p, pl.ds(c0, 16)] * float(ww) + 0.5
                    ty = xy_v[8 * l + 2 * p + 1, pl.ds(c0, 16)] * float(hh) + 0.5
                    w = aw_v[4 * l + p, pl.ds(c0, 16)]
                    x0 = tx.astype(jnp.int32)
                    y0 = ty.astype(jnp.int32)
                    fx = tx - x0.astype(jnp.float32)
                    fy = ty - y0.astype(jnp.float32)
                    ay1 = fy * w
                    ay0 = w - ay1
                    r0 = y0 * wp + base + x0
                    corners = (
                        (r0, (1.0 - fx) * ay0),
                        (r0 + 1, fx * ay0),
                        (r0 + wp, (1.0 - fx) * ay1),
                        (r0 + wp + 1, fx * ay1),
                    )
                    for adr, wc in corners:
                        cur = adr
                        for dd in range(_DH):
                            g = plsc.load_gather(table_v, [cur])
                            acc[dd] = acc[dd] + g * wc
                            if dd + 1 < _DH:
                                cur = cur + _SP
            for dd in range(_DH):
                out_v[dd, pl.ds(c0, 16)] = acc[dd]
            return carry2

        lax.fori_loop(0, _VB, iv_body, 0)
        pltpu.sync_copy(out_v, out_hbm.at[wid, :, pl.ds(q0, _QB)])
        return carry

    lax.fori_loop(0, _NCH, chunk_body, 0)


@jax.jit
def _run(table, xy, aw):
    mesh = plsc.VectorSubcoreMesh(core_axis_name="c", subcore_axis_name="s")
    f = functools.partial(
        pl.kernel,
        mesh=mesh,
        out_type=jax.ShapeDtypeStruct((_NW, _DH, _Q), jnp.float32),
        scratch_types=[
            pltpu.VMEM((_DH * _SP,), jnp.float32),
            pltpu.VMEM((2 * _L * _P, _QB), jnp.float32),
            pltpu.VMEM((_L * _P, _QB), jnp.float32),
            pltpu.VMEM((_DH, _QB), jnp.float32),
        ],
        compiler_params=pltpu.CompilerParams(
            use_tc_tiling_on_sc=False, needs_layout_passes=False),
    )(_body)
    return f(table, xy, aw)


def kernel(value, value_spatial_shapes, value_level_start_index,
           sampling_locations, attention_weights, im2col_step):
    B, S, H, D = value.shape
    Q = sampling_locations.shape[1]
    vt = jnp.transpose(value, (0, 2, 3, 1)).reshape(B * H, D, S)
    parts = []
    for l, (hh, ww) in enumerate(_SRC_SHAPES):
        lvl = lax.slice_in_dim(vt, _STARTS[l], _STARTS[l] + hh * ww, axis=2)
        lvl = lvl.reshape(B * H, D, hh, ww)
        lvl = jnp.pad(lvl, ((0, 0), (0, 0), (1, 1), (1, 1)))
        parts.append(lvl.reshape(B * H, D, (hh + 2) * (ww + 2)))
    table = jnp.concatenate(parts, axis=2).reshape(_NW, _DH * _SP)
    xy = jnp.transpose(sampling_locations, (0, 2, 3, 4, 5, 1)).reshape(
        B * H, _L * _P * 2, Q)
    aw = jnp.transpose(attention_weights, (0, 2, 3, 4, 1)).reshape(
        B * H, _L * _P, Q)
    out = _run(table, xy, aw)
    out = out.reshape(B, H, D, Q)
    return jnp.transpose(out, (0, 3, 1, 2)).reshape(B, Q, H * D)
